# baseline (device time: 26223 ns/iter reference)
import jax
import jax.numpy as jnp
from jax import lax
from jax.experimental import pallas as pl
from jax.experimental.pallas import tpu as pltpu

N_DEV = 4
N_LAYERS = 3


def kernel(x, Win0, Wout0, Win1, Wout1, Win2, Wout2):
    m, d = x.shape

    def body(x_ref, win0_ref, wout0_ref, win1_ref, wout1_ref, win2_ref,
             wout2_ref, out_ref, comm_ref, send_sems, recv_sems):
        my_pos = lax.axis_index("i")

        barrier_sem = pltpu.get_barrier_semaphore()
        for off in range(1, N_DEV):
            peer = (my_pos + off) % N_DEV
            pl.semaphore_signal(
                barrier_sem, inc=1,
                device_id=(peer,), device_id_type=pl.DeviceIdType.MESH,
            )
        pl.semaphore_wait(barrier_sem, N_DEV - 1)

        xb = x_ref[...].astype(jnp.bfloat16)
        wins = [win0_ref, win1_ref, win2_ref]
        wouts = [wout0_ref, wout1_ref, wout2_ref]

        for k in range(N_LAYERS):
            wi = wins[k][...].astype(jnp.bfloat16)
            wo = wouts[k][...].astype(jnp.bfloat16)
            h = jnp.dot(xb, wi, preferred_element_type=jnp.float32)
            h = jnp.maximum(h, 0.0).astype(jnp.bfloat16)
            p = jnp.dot(h, wo, preferred_element_type=jnp.float32)

            comm_ref[k, my_pos] = p.astype(jnp.bfloat16)
            sends = []
            for off in range(1, N_DEV):
                peer = (my_pos + off) % N_DEV
                rdma = pltpu.make_async_remote_copy(
                    src_ref=comm_ref.at[k, my_pos],
                    dst_ref=comm_ref.at[k, my_pos],
                    send_sem=send_sems.at[k, off - 1],
                    recv_sem=recv_sems.at[k, my_pos],
                    device_id=(peer,),
                    device_id_type=pl.DeviceIdType.MESH,
                )
                rdma.start()
                sends.append(rdma)

            for off in range(1, N_DEV):
                sender = (my_pos + off) % N_DEV
                recv = pltpu.make_async_remote_copy(
                    src_ref=comm_ref.at[k, sender],
                    dst_ref=comm_ref.at[k, sender],
                    send_sem=send_sems.at[k, off - 1],
                    recv_sem=recv_sems.at[k, sender],
                    device_id=(my_pos,),
                    device_id_type=pl.DeviceIdType.MESH,
                )
                recv.wait_recv()
            for rdma in sends:
                rdma.wait_send()

            acc = jnp.sum(comm_ref[k].astype(jnp.float32), axis=0)
            if k < N_LAYERS - 1:
                xb = acc.astype(jnp.bfloat16)
            else:
                out_ref[...] = acc

    return pl.pallas_call(
        body,
        out_shape=jax.ShapeDtypeStruct((m, d), jnp.float32),
        in_specs=[pl.BlockSpec(memory_space=pltpu.VMEM)] * 7,
        out_specs=pl.BlockSpec(memory_space=pltpu.VMEM),
        scratch_shapes=[
            pltpu.VMEM((N_LAYERS, N_DEV, m, d), jnp.bfloat16),
            pltpu.SemaphoreType.DMA((N_LAYERS, N_DEV - 1)),
            pltpu.SemaphoreType.DMA((N_LAYERS, N_DEV)),
        ],
        compiler_params=pltpu.CompilerParams(collective_id=0),
    )(x, Win0, Wout0, Win1, Wout1, Win2, Wout2)
